# Initial kernel scaffold; baseline (speedup 1.0000x reference)
import jax
import jax.numpy as jnp
from jax.experimental import pallas as pl


def kernel(full_graph_census, full_graph_population_t, ethnicity, node_idx, edge_index, edge_weight, enc_W1, enc_b1, enc_g1, enc_be1, enc_W2, enc_b2, enc_g2, enc_be2, gcn_W0, gcn_b0, gcn_g0, gcn_be0, gcn_W1, gcn_b1, gcn_g1, gcn_be1, gcn_W2, gcn_b2, gcn_g2, gcn_be2, eth_table, pred_W1, pred_b1, pred_g1, pred_be1, pred_W2, pred_b2, pred_g2, pred_be2, pred_W3, pred_b3):
    raise NotImplementedError("write your pallas kernel here")


# speedup vs baseline: 8.8960x; 8.8960x over previous
import dataclasses
import functools

import jax
import jax.numpy as jnp
from jax import lax
from jax.experimental import pallas as pl
from jax.experimental.pallas import tpu as pltpu
from jax.experimental.pallas import tpu_sc as plsc

N = 10000
E = 320000
NC = 64
NE = 16
B = 4096
D = 128
ED = 32

_SC_CORES = 2
_SC_SUBCORES = 16
_NW = _SC_CORES * _SC_SUBCORES
_EPT = E // _NW
_W = 80
_NGW = _EPT // _W
_RPS = N // _SC_SUBCORES
_ZR = 125
_BPW = B // _NW

_mesh = plsc.VectorSubcoreMesh(core_axis_name="c", subcore_axis_name="s")

_SC_PARAMS = pltpu.CompilerParams()
if "needs_layout_passes" in pltpu.CompilerParams.__dataclass_fields__:
    _SC_PARAMS = dataclasses.replace(_SC_PARAMS, needs_layout_passes=False)

_HIGH = jax.lax.Precision.HIGHEST


@functools.partial(
    pl.kernel,
    out_type=jax.ShapeDtypeStruct((_NW * N,), jnp.float32),
    mesh=_mesh,
    compiler_params=_SC_PARAMS,
    scratch_types=[
        pltpu.VMEM((_EPT,), jnp.int32),
        pltpu.VMEM((_EPT,), jnp.float32),
        pltpu.VMEM((N,), jnp.float32),
    ],
)
def _deg_kernel(d_hbm, w_hbm, out_hbm, didx, wvals, deg):
    cid = lax.axis_index("c")
    sid = lax.axis_index("s")
    wid = cid * _SC_SUBCORES + sid
    base = pl.multiple_of(wid * _EPT, 8)
    pltpu.sync_copy(d_hbm.at[pl.ds(base, _EPT)], didx)
    pltpu.sync_copy(w_hbm.at[pl.ds(base, _EPT)], wvals)

    @pl.loop(0, N, step=16)
    def _(i):
        deg[pl.ds(i, 16)] = jnp.zeros((16,), jnp.float32)

    @pl.loop(0, _EPT, step=16)
    def _(i):
        idx = didx[pl.ds(i, 16)]
        vals = wvals[pl.ds(i, 16)]
        plsc.addupdate_scatter(deg, [idx], vals)

    pltpu.sync_copy(deg, out_hbm.at[pl.ds(pl.multiple_of(wid * N, 8), N)])


@functools.partial(
    pl.kernel,
    out_type=jax.ShapeDtypeStruct((_SC_CORES, _SC_SUBCORES, _RPS, D),
                                  jnp.float32),
    mesh=_mesh,
    compiler_params=_SC_PARAMS,
    scratch_types=[
        pltpu.VMEM((_EPT,), jnp.int32),
        pltpu.VMEM((_EPT,), jnp.float32),
        pltpu.VMEM((_W,), jnp.int32),
        pltpu.VMEM((_W, D), jnp.float32),
        pltpu.VMEM((_ZR, D), jnp.float32),
        pltpu.VMEM_SHARED((N, D), jnp.float32),
    ],
)
def _agg_kernel(xws_hbm, s_hbm, d_hbm, w_hbm, out_hbm,
                sidx, wvals, didx, rows, zbuf, acc):
    cid = lax.axis_index("c")
    sid = lax.axis_index("s")
    wid = cid * _SC_SUBCORES + sid
    base = pl.multiple_of(wid * _EPT, 8)

    @pl.loop(0, _ZR)
    def _(r):
        for c in range(D // 16):
            zbuf[r, pl.ds(c * 16, 16)] = jnp.zeros((16,), jnp.float32)

    for k in range(_RPS // _ZR):
        pltpu.sync_copy(zbuf, acc.at[pl.ds(sid * _RPS + k * _ZR, _ZR)])
    plsc.subcore_barrier()

    pltpu.sync_copy(s_hbm.at[pl.ds(base, _EPT)], sidx)
    pltpu.sync_copy(w_hbm.at[pl.ds(base, _EPT)], wvals)

    @pl.loop(0, _NGW)
    def _(g):
        off = pl.multiple_of(g * _W, 16)
        pltpu.sync_copy(d_hbm.at[pl.ds(base + off, _W)], didx)
        pltpu.sync_copy(xws_hbm.at[sidx.at[pl.ds(off, _W)]], rows)

        @pl.loop(0, _W)
        def _(e):
            wv = plsc.load_gather(wvals, [jnp.full((16,), off + e, jnp.int32)])
            for c in range(D // 16):
                sl = pl.ds(c * 16, 16)
                rows[e, sl] = rows[e, sl] * wv

        pltpu.sync_copy(rows, acc.at[didx], add=True)

    plsc.subcore_barrier()
    pltpu.sync_copy(acc.at[pl.ds(sid * _RPS, _RPS)], out_hbm.at[cid, sid])


@functools.partial(
    pl.kernel,
    out_type=jax.ShapeDtypeStruct((B, D), jnp.float32),
    mesh=_mesh,
    compiler_params=_SC_PARAMS,
    scratch_types=[
        pltpu.VMEM((_BPW,), jnp.int32),
        pltpu.VMEM((_BPW, D), jnp.float32),
        pltpu.SemaphoreType.DMA,
    ],
)
def _bgather_kernel(x_hbm, idx_hbm, out_hbm, idxv, rowsv, sem):
    wid = lax.axis_index("s") * _SC_CORES + lax.axis_index("c")
    base = pl.multiple_of(wid * _BPW, 8)
    pltpu.sync_copy(idx_hbm.at[pl.ds(base, _BPW)], idxv)
    pltpu.async_copy(x_hbm.at[idxv], rowsv, sem).wait()
    pltpu.sync_copy(rowsv, out_hbm.at[pl.ds(base, _BPW)])


def _ln(x, g, b):
    mu = jnp.mean(x, axis=-1, keepdims=True)
    v = jnp.mean((x - mu) ** 2, axis=-1, keepdims=True)
    return (x - mu) / jnp.sqrt(v + 1e-5) * g + b


def _dinv_body(parts_ref, out_ref):
    deg = 1.0 + jnp.sum(parts_ref[...], axis=0, keepdims=True)
    out_ref[...] = lax.rsqrt(deg)


def _encoder_body(census_ref, pop_ref, w1a_ref, w1b_ref, b1_ref, g1_ref,
                  be1_ref, w2_ref, b2_ref, g2_ref, be2_ref, out_ref):
    other = jnp.sum(pop_ref[...], axis=1, keepdims=True)
    h = (jnp.dot(census_ref[...], w1a_ref[...], precision=_HIGH)
         + other * w1b_ref[...] + b1_ref[...])
    h = jax.nn.relu(_ln(h, g1_ref[...], be1_ref[...]))
    h = jnp.dot(h, w2_ref[...], precision=_HIGH) + b2_ref[...]
    out_ref[...] = jax.nn.relu(_ln(h, g2_ref[...], be2_ref[...]))


def _pre_body(x_ref, w_ref, dinv_ref, xw_ref, xws_ref):
    xw = jnp.dot(x_ref[...], w_ref[...], precision=_HIGH)
    xw_ref[...] = xw
    xws_ref[...] = xw * dinv_ref[...]


def _post_body(parts_ref, xw_ref, dinv_ref, b_ref, g_ref, be_ref, xin_ref,
               out_ref, *, residual):
    dinv = dinv_ref[...]
    agg = ((parts_ref[0] + parts_ref[1]) * dinv
           + xw_ref[...] * (dinv * dinv) + b_ref[...])
    y = jax.nn.relu(_ln(agg, g_ref[...], be_ref[...]))
    if residual:
        y = y + xin_ref[...]
    out_ref[...] = y


def _pred_body(bs_ref, eth_ref, table_ref, w1a_ref, w1b_ref, b1_ref, g1_ref,
               be1_ref, w2_ref, b2_ref, g2_ref, be2_ref, w3_ref, b3_ref,
               out_ref):
    onehot = (lax.broadcasted_iota(jnp.int32, (B, NE), 1)
              == eth_ref[...]).astype(jnp.float32)
    ee = jnp.dot(onehot, table_ref[...], precision=_HIGH)
    c1 = (jnp.dot(bs_ref[...], w1a_ref[...], precision=_HIGH)
          + jnp.dot(ee, w1b_ref[...], precision=_HIGH) + b1_ref[...])
    p = jax.nn.relu(_ln(c1, g1_ref[...], be1_ref[...]))
    p = jnp.dot(p, w2_ref[...], precision=_HIGH) + b2_ref[...]
    p = jax.nn.relu(_ln(p, g2_ref[...], be2_ref[...]))
    p = jnp.dot(p, w3_ref[...], precision=_HIGH) + b3_ref[...]
    out_ref[...] = jax.nn.softplus(p)


def _f32(shape):
    return jax.ShapeDtypeStruct(shape, jnp.float32)


_dinv_call = pl.pallas_call(_dinv_body, out_shape=_f32((1, N)))
_encoder_call = pl.pallas_call(_encoder_body, out_shape=_f32((N, D)))
_pre_call = pl.pallas_call(_pre_body, out_shape=(_f32((N, D)), _f32((N, D))))
_post_res_call = pl.pallas_call(
    functools.partial(_post_body, residual=True), out_shape=_f32((N, D)))
_post_nores_call = pl.pallas_call(
    functools.partial(_post_body, residual=False), out_shape=_f32((N, D)))
_pred_call = pl.pallas_call(_pred_body, out_shape=_f32((B, 1)))


def kernel(full_graph_census, full_graph_population_t, ethnicity, node_idx,
           edge_index, edge_weight,
           enc_W1, enc_b1, enc_g1, enc_be1, enc_W2, enc_b2, enc_g2, enc_be2,
           gcn_W0, gcn_b0, gcn_g0, gcn_be0, gcn_W1, gcn_b1, gcn_g1, gcn_be1,
           gcn_W2, gcn_b2, gcn_g2, gcn_be2, eth_table,
           pred_W1, pred_b1, pred_g1, pred_be1, pred_W2, pred_b2, pred_g2,
           pred_be2, pred_W3, pred_b3):
    s = edge_index[0].astype(jnp.int32)
    d = edge_index[1].astype(jnp.int32)
    w = edge_weight

    row = lambda a: a.reshape(1, -1)

    deg_parts = _deg_kernel(d, w).reshape(_NW, N)
    dinv = _dinv_call(deg_parts).reshape(N, 1)

    h = _encoder_call(full_graph_census, full_graph_population_t,
                      enc_W1[:NC], enc_W1[NC:], row(enc_b1), row(enc_g1),
                      row(enc_be1), enc_W2, row(enc_b2), row(enc_g2),
                      row(enc_be2))

    layers = [
        (gcn_W0, gcn_b0, gcn_g0, gcn_be0),
        (gcn_W1, gcn_b1, gcn_g1, gcn_be1),
        (gcn_W2, gcn_b2, gcn_g2, gcn_be2),
    ]
    x = h
    for i, (Wt, bb, g, be) in enumerate(layers):
        xw, xws = _pre_call(x, Wt, dinv)
        parts = _agg_kernel(xws, s, d, w).reshape(_SC_CORES, N, D)
        post = _post_res_call if i > 0 else _post_nores_call
        x = post(parts, xw, dinv, row(bb), row(g), row(be), x)

    bs = _bgather_kernel(x, node_idx.astype(jnp.int32))
    out = _pred_call(bs, ethnicity.astype(jnp.int32).reshape(B, 1), eth_table,
                     pred_W1[:D], pred_W1[D:], row(pred_b1), row(pred_g1),
                     row(pred_be1), pred_W2, row(pred_b2), row(pred_g2),
                     row(pred_be2), pred_W3, row(pred_b3))
    return out.reshape(B)


# speedup vs baseline: 11.7076x; 1.3161x over previous
import dataclasses
import functools

import jax
import jax.numpy as jnp
from jax import lax
from jax.experimental import pallas as pl
from jax.experimental.pallas import tpu as pltpu
from jax.experimental.pallas import tpu_sc as plsc

N = 10000
E = 320000
NC = 64
NE = 16
B = 4096
D = 128
ED = 32

_SC_CORES = 2
_SC_SUBCORES = 16
_NW = _SC_CORES * _SC_SUBCORES
_EPT = E // _NW
_W = 80
_NGW = _EPT // _W
_RPS = N // _SC_SUBCORES
_ZR = 125
_BPW = B // _NW

_mesh = plsc.VectorSubcoreMesh(core_axis_name="c", subcore_axis_name="s")

_SC_PARAMS = pltpu.CompilerParams()
if "needs_layout_passes" in pltpu.CompilerParams.__dataclass_fields__:
    _SC_PARAMS = dataclasses.replace(_SC_PARAMS, needs_layout_passes=False)

_HIGH = jax.lax.Precision.HIGHEST


@functools.partial(
    pl.kernel,
    out_type=jax.ShapeDtypeStruct((_NW * N,), jnp.float32),
    mesh=_mesh,
    compiler_params=_SC_PARAMS,
    scratch_types=[
        pltpu.VMEM((_EPT,), jnp.int32),
        pltpu.VMEM((_EPT,), jnp.float32),
        pltpu.VMEM((N,), jnp.float32),
    ],
)
def _deg_kernel(d_hbm, w_hbm, out_hbm, didx, wvals, deg):
    cid = lax.axis_index("c")
    sid = lax.axis_index("s")
    wid = cid * _SC_SUBCORES + sid
    base = pl.multiple_of(wid * _EPT, 8)
    pltpu.sync_copy(d_hbm.at[pl.ds(base, _EPT)], didx)
    pltpu.sync_copy(w_hbm.at[pl.ds(base, _EPT)], wvals)

    @pl.loop(0, N, step=16)
    def _(i):
        deg[pl.ds(i, 16)] = jnp.zeros((16,), jnp.float32)

    @pl.loop(0, _EPT, step=16)
    def _(i):
        idx = didx[pl.ds(i, 16)]
        vals = wvals[pl.ds(i, 16)]
        plsc.addupdate_scatter(deg, [idx], vals)

    pltpu.sync_copy(deg, out_hbm.at[pl.ds(pl.multiple_of(wid * N, 8), N)])


_NBUF = 2


@functools.partial(
    pl.kernel,
    out_type=jax.ShapeDtypeStruct((_SC_CORES, _SC_SUBCORES, _RPS, D),
                                  jnp.float32),
    mesh=_mesh,
    compiler_params=_SC_PARAMS,
    scratch_types=[
        pltpu.VMEM((_EPT,), jnp.int32),
        pltpu.VMEM((_EPT,), jnp.float32),
        [pltpu.VMEM((_W,), jnp.int32) for _ in range(_NBUF)],
        [pltpu.VMEM((_W, D), jnp.float32) for _ in range(_NBUF)],
        pltpu.VMEM_SHARED((N, D), jnp.float32),
        [pltpu.SemaphoreType.DMA for _ in range(3 * _NBUF)],
    ],
)
def _agg_kernel(xws_hbm, s_hbm, d_hbm, w_hbm, out_hbm,
                sidx, wvals, dbufs, bufs, acc, sems):
    cid = lax.axis_index("c")
    sid = lax.axis_index("s")
    wid = cid * _SC_SUBCORES + sid
    base = pl.multiple_of(wid * _EPT, 8)
    gsems = sems[:_NBUF]
    ssems = sems[_NBUF:2 * _NBUF]
    dsems = sems[2 * _NBUF:]

    @pl.loop(0, _W)
    def _(r):
        for c in range(D // 16):
            bufs[0][r, pl.ds(c * 16, 16)] = jnp.zeros((16,), jnp.float32)

    for k in range(7):
        pltpu.sync_copy(bufs[0], acc.at[pl.ds(sid * _RPS + k * _W, _W)])
    pltpu.sync_copy(bufs[0].at[pl.ds(0, _RPS - 7 * _W)],
                    acc.at[pl.ds(sid * _RPS + 7 * _W, _RPS - 7 * _W)])
    plsc.subcore_barrier()

    pltpu.sync_copy(s_hbm.at[pl.ds(base, _EPT)], sidx)
    pltpu.sync_copy(w_hbm.at[pl.ds(base, _EPT)], wvals)

    def start_gather(w, b):
        off = pl.multiple_of(w * _W, 16)
        pltpu.async_copy(xws_hbm.at[sidx.at[pl.ds(off, _W)]], bufs[b],
                         gsems[b])
        pltpu.async_copy(d_hbm.at[pl.ds(base + off, _W)], dbufs[b], dsems[b])

    def wait_gather(b):
        pltpu.make_async_copy(xws_hbm.at[pl.ds(0, _W)], bufs[b],
                              gsems[b]).wait()

    def scale(w, b):
        off = w * _W

        @pl.loop(0, _W)
        def _(e):
            wv = plsc.load_gather(wvals, [jnp.full((16,), off + e, jnp.int32)])
            for c in range(D // 16):
                sl = pl.ds(c * 16, 16)
                bufs[b][e, sl] = bufs[b][e, sl] * wv

    def start_scatter(w, b):
        pltpu.make_async_copy(d_hbm.at[pl.ds(0, _W)], dbufs[b],
                              dsems[b]).wait()
        pltpu.async_copy(bufs[b], acc.at[dbufs[b]], ssems[b], add=True)

    def wait_scatter(b):
        pltpu.make_async_copy(bufs[b], acc.at[pl.ds(0, _W)], ssems[b]).wait()

    def window(w, b, wait_prev, prefetch):
        wait_gather(b)
        scale(w, b)
        start_scatter(w, b)
        pb = (b - 1) % _NBUF
        if wait_prev:
            wait_scatter(pb)
        if prefetch:
            start_gather(w + _NBUF - 1, pb)

    for j in range(_NBUF - 1):
        start_gather(j, j)
    window(0, 0, wait_prev=False, prefetch=True)

    @pl.loop(1, _NGW - _NBUF, step=_NBUF)
    def _(w0):
        for j in range(_NBUF):
            window(w0 + j, (1 + j) % _NBUF, wait_prev=True, prefetch=True)

    for w in range(_NGW - _NBUF, _NGW):
        window(w, w % _NBUF, wait_prev=True, prefetch=(w + _NBUF - 1 < _NGW))
    wait_scatter((_NGW - 1) % _NBUF)

    plsc.subcore_barrier()
    pltpu.sync_copy(acc.at[pl.ds(sid * _RPS, _RPS)], out_hbm.at[cid, sid])


@functools.partial(
    pl.kernel,
    out_type=jax.ShapeDtypeStruct((B, D), jnp.float32),
    mesh=_mesh,
    compiler_params=_SC_PARAMS,
    scratch_types=[
        pltpu.VMEM((_BPW,), jnp.int32),
        pltpu.VMEM((_BPW, D), jnp.float32),
        pltpu.SemaphoreType.DMA,
    ],
)
def _bgather_kernel(x_hbm, idx_hbm, out_hbm, idxv, rowsv, sem):
    wid = lax.axis_index("s") * _SC_CORES + lax.axis_index("c")
    base = pl.multiple_of(wid * _BPW, 8)
    pltpu.sync_copy(idx_hbm.at[pl.ds(base, _BPW)], idxv)
    pltpu.async_copy(x_hbm.at[idxv], rowsv, sem).wait()
    pltpu.sync_copy(rowsv, out_hbm.at[pl.ds(base, _BPW)])


def _ln(x, g, b):
    mu = jnp.mean(x, axis=-1, keepdims=True)
    v = jnp.mean((x - mu) ** 2, axis=-1, keepdims=True)
    return (x - mu) / jnp.sqrt(v + 1e-5) * g + b


def _dinv_body(parts_ref, out_ref):
    deg = 1.0 + jnp.sum(parts_ref[...], axis=0, keepdims=True)
    out_ref[...] = lax.rsqrt(deg)


def _encoder_body(census_ref, pop_ref, w1a_ref, w1b_ref, b1_ref, g1_ref,
                  be1_ref, w2_ref, b2_ref, g2_ref, be2_ref, out_ref):
    other = jnp.sum(pop_ref[...], axis=1, keepdims=True)
    h = (jnp.dot(census_ref[...], w1a_ref[...], precision=_HIGH)
         + other * w1b_ref[...] + b1_ref[...])
    h = jax.nn.relu(_ln(h, g1_ref[...], be1_ref[...]))
    h = jnp.dot(h, w2_ref[...], precision=_HIGH) + b2_ref[...]
    out_ref[...] = jax.nn.relu(_ln(h, g2_ref[...], be2_ref[...]))


def _pre_body(x_ref, w_ref, dinv_ref, xw_ref, xws_ref):
    xw = jnp.dot(x_ref[...], w_ref[...], precision=_HIGH)
    xw_ref[...] = xw
    xws_ref[...] = xw * dinv_ref[...]


def _post_body(parts_ref, xw_ref, dinv_ref, b_ref, g_ref, be_ref, xin_ref,
               out_ref, *, residual):
    dinv = dinv_ref[...]
    agg = ((parts_ref[0] + parts_ref[1]) * dinv
           + xw_ref[...] * (dinv * dinv) + b_ref[...])
    y = jax.nn.relu(_ln(agg, g_ref[...], be_ref[...]))
    if residual:
        y = y + xin_ref[...]
    out_ref[...] = y


def _pred_body(bs_ref, eth_ref, table_ref, w1a_ref, w1b_ref, b1_ref, g1_ref,
               be1_ref, w2_ref, b2_ref, g2_ref, be2_ref, w3_ref, b3_ref,
               out_ref):
    onehot = (lax.broadcasted_iota(jnp.int32, (B, NE), 1)
              == eth_ref[...]).astype(jnp.float32)
    ee = jnp.dot(onehot, table_ref[...], precision=_HIGH)
    c1 = (jnp.dot(bs_ref[...], w1a_ref[...], precision=_HIGH)
          + jnp.dot(ee, w1b_ref[...], precision=_HIGH) + b1_ref[...])
    p = jax.nn.relu(_ln(c1, g1_ref[...], be1_ref[...]))
    p = jnp.dot(p, w2_ref[...], precision=_HIGH) + b2_ref[...]
    p = jax.nn.relu(_ln(p, g2_ref[...], be2_ref[...]))
    p = jnp.dot(p, w3_ref[...], precision=_HIGH) + b3_ref[...]
    out_ref[...] = jax.nn.softplus(p)


def _f32(shape):
    return jax.ShapeDtypeStruct(shape, jnp.float32)


_dinv_call = pl.pallas_call(_dinv_body, out_shape=_f32((1, N)))
_encoder_call = pl.pallas_call(_encoder_body, out_shape=_f32((N, D)))
_pre_call = pl.pallas_call(_pre_body, out_shape=(_f32((N, D)), _f32((N, D))))
_post_res_call = pl.pallas_call(
    functools.partial(_post_body, residual=True), out_shape=_f32((N, D)))
_post_nores_call = pl.pallas_call(
    functools.partial(_post_body, residual=False), out_shape=_f32((N, D)))
_pred_call = pl.pallas_call(_pred_body, out_shape=_f32((B, 1)))


def kernel(full_graph_census, full_graph_population_t, ethnicity, node_idx,
           edge_index, edge_weight,
           enc_W1, enc_b1, enc_g1, enc_be1, enc_W2, enc_b2, enc_g2, enc_be2,
           gcn_W0, gcn_b0, gcn_g0, gcn_be0, gcn_W1, gcn_b1, gcn_g1, gcn_be1,
           gcn_W2, gcn_b2, gcn_g2, gcn_be2, eth_table,
           pred_W1, pred_b1, pred_g1, pred_be1, pred_W2, pred_b2, pred_g2,
           pred_be2, pred_W3, pred_b3):
    s = edge_index[0].astype(jnp.int32)
    d = edge_index[1].astype(jnp.int32)
    w = edge_weight

    row = lambda a: a.reshape(1, -1)

    deg_parts = _deg_kernel(d, w).reshape(_NW, N)
    dinv = _dinv_call(deg_parts).reshape(N, 1)

    h = _encoder_call(full_graph_census, full_graph_population_t,
                      enc_W1[:NC], enc_W1[NC:], row(enc_b1), row(enc_g1),
                      row(enc_be1), enc_W2, row(enc_b2), row(enc_g2),
                      row(enc_be2))

    layers = [
        (gcn_W0, gcn_b0, gcn_g0, gcn_be0),
        (gcn_W1, gcn_b1, gcn_g1, gcn_be1),
        (gcn_W2, gcn_b2, gcn_g2, gcn_be2),
    ]
    x = h
    for i, (Wt, bb, g, be) in enumerate(layers):
        xw, xws = _pre_call(x, Wt, dinv)
        parts = _agg_kernel(xws, s, d, w).reshape(_SC_CORES, N, D)
        post = _post_res_call if i > 0 else _post_nores_call
        x = post(parts, xw, dinv, row(bb), row(g), row(be), x)

    bs = _bgather_kernel(x, node_idx.astype(jnp.int32))
    out = _pred_call(bs, ethnicity.astype(jnp.int32).reshape(B, 1), eth_table,
                     pred_W1[:D], pred_W1[D:], row(pred_b1), row(pred_g1),
                     row(pred_be1), pred_W2, row(pred_b2), row(pred_g2),
                     row(pred_be2), pred_W3, row(pred_b3))
    return out.reshape(B)


# speedup vs baseline: 17.3179x; 1.4792x over previous
import dataclasses
import functools

import jax
import jax.numpy as jnp
from jax import lax
from jax.experimental import pallas as pl
from jax.experimental.pallas import tpu as pltpu
from jax.experimental.pallas import tpu_sc as plsc

N = 10000
E = 320000
NC = 64
NE = 16
B = 4096
D = 128
ED = 32

_SC_CORES = 2
_SC_SUBCORES = 16
_NW = _SC_CORES * _SC_SUBCORES
_EPT = E // _NW
_W = 72
_NGW = 140
_EPTA = _NGW * _W
_EPAD = _NW * _EPTA
_RPS = N // _SC_SUBCORES
_ZR = 125
_BPW = B // _NW

_mesh = plsc.VectorSubcoreMesh(core_axis_name="c", subcore_axis_name="s")

_SC_PARAMS = pltpu.CompilerParams()
if "needs_layout_passes" in pltpu.CompilerParams.__dataclass_fields__:
    _SC_PARAMS = dataclasses.replace(_SC_PARAMS, needs_layout_passes=False)

_HIGH = jax.lax.Precision.HIGHEST


@functools.partial(
    pl.kernel,
    out_type=jax.ShapeDtypeStruct((_NW * N,), jnp.float32),
    mesh=_mesh,
    compiler_params=_SC_PARAMS,
    scratch_types=[
        pltpu.VMEM((_EPT,), jnp.int32),
        pltpu.VMEM((_EPT,), jnp.float32),
        pltpu.VMEM((N,), jnp.float32),
    ],
)
def _deg_kernel(d_hbm, w_hbm, out_hbm, didx, wvals, deg):
    cid = lax.axis_index("c")
    sid = lax.axis_index("s")
    wid = cid * _SC_SUBCORES + sid
    base = pl.multiple_of(wid * _EPT, 8)
    pltpu.sync_copy(d_hbm.at[pl.ds(base, _EPT)], didx)
    pltpu.sync_copy(w_hbm.at[pl.ds(base, _EPT)], wvals)

    @pl.loop(0, N, step=16)
    def _(i):
        deg[pl.ds(i, 16)] = jnp.zeros((16,), jnp.float32)

    @pl.loop(0, _EPT, step=16)
    def _(i):
        idx = didx[pl.ds(i, 16)]
        vals = wvals[pl.ds(i, 16)]
        plsc.addupdate_scatter(deg, [idx], vals)

    pltpu.sync_copy(deg, out_hbm.at[pl.ds(pl.multiple_of(wid * N, 8), N)])


_NBUF = 3


@functools.partial(
    pl.kernel,
    out_type=jax.ShapeDtypeStruct((_SC_CORES, _SC_SUBCORES, _RPS, D),
                                  jnp.float32),
    mesh=_mesh,
    compiler_params=_SC_PARAMS,
    scratch_types=[
        pltpu.VMEM((_EPTA,), jnp.int32),
        pltpu.VMEM((_EPTA,), jnp.float32),
        [pltpu.VMEM((_W,), jnp.int32) for _ in range(_NBUF)],
        [pltpu.VMEM((_W, D), jnp.float32) for _ in range(_NBUF)],
        pltpu.VMEM_SHARED((N, D), jnp.float32),
        [pltpu.SemaphoreType.DMA for _ in range(3 * _NBUF)],
    ],
)
def _agg_kernel(xws_hbm, s_hbm, d_hbm, w_hbm, out_hbm,
                sidx, wvals, dbufs, bufs, acc, sems):
    cid = lax.axis_index("c")
    sid = lax.axis_index("s")
    wid = cid * _SC_SUBCORES + sid
    base = pl.multiple_of(wid * _EPTA, 8)
    gsems = sems[:_NBUF]
    ssems = sems[_NBUF:2 * _NBUF]
    dsems = sems[2 * _NBUF:]

    @pl.loop(0, _W)
    def _(r):
        for c in range(D // 16):
            bufs[0][r, pl.ds(c * 16, 16)] = jnp.zeros((16,), jnp.float32)

    nz = _RPS // _W
    for k in range(nz):
        pltpu.sync_copy(bufs[0], acc.at[pl.ds(sid * _RPS + k * _W, _W)])
    pltpu.sync_copy(bufs[0].at[pl.ds(0, _RPS - nz * _W)],
                    acc.at[pl.ds(sid * _RPS + nz * _W, _RPS - nz * _W)])
    plsc.subcore_barrier()

    pltpu.sync_copy(s_hbm.at[pl.ds(base, _EPTA)], sidx)
    pltpu.sync_copy(w_hbm.at[pl.ds(base, _EPTA)], wvals)

    def start_gather(w, b):
        off = pl.multiple_of(w * _W, 8)
        pltpu.async_copy(xws_hbm.at[sidx.at[pl.ds(off, _W)]], bufs[b],
                         gsems[b])
        pltpu.async_copy(d_hbm.at[pl.ds(base + off, _W)], dbufs[b], dsems[b])

    def wait_gather(b):
        pltpu.make_async_copy(xws_hbm.at[pl.ds(0, _W)], bufs[b],
                              gsems[b]).wait()

    def scale(w, b):
        off = w * _W

        @pl.loop(0, _W)
        def _(e):
            wv = plsc.load_gather(wvals, [jnp.full((16,), off + e, jnp.int32)])
            for c in range(D // 16):
                sl = pl.ds(c * 16, 16)
                bufs[b][e, sl] = bufs[b][e, sl] * wv

    def start_scatter(w, b):
        pltpu.make_async_copy(d_hbm.at[pl.ds(0, _W)], dbufs[b],
                              dsems[b]).wait()
        pltpu.async_copy(bufs[b], acc.at[dbufs[b]], ssems[b], add=True)

    def wait_scatter(b):
        pltpu.make_async_copy(bufs[b], acc.at[pl.ds(0, _W)], ssems[b]).wait()

    def window(w, b, wait_prev, prefetch):
        wait_gather(b)
        scale(w, b)
        start_scatter(w, b)
        pb = (b - 1) % _NBUF
        if wait_prev:
            wait_scatter(pb)
        if prefetch:
            start_gather(w + _NBUF - 1, pb)

    _SE = _NGW - 4

    for j in range(_NBUF - 1):
        start_gather(j, j)
    window(0, 0, wait_prev=False, prefetch=True)

    @pl.loop(1, _SE, step=_NBUF)
    def _(w0):
        for j in range(_NBUF):
            window(w0 + j, (1 + j) % _NBUF, wait_prev=True, prefetch=True)

    for w in range(_SE, _NGW):
        window(w, w % _NBUF, wait_prev=True, prefetch=(w + _NBUF - 1 < _NGW))
    wait_scatter((_NGW - 1) % _NBUF)

    plsc.subcore_barrier()
    pltpu.sync_copy(acc.at[pl.ds(sid * _RPS, _RPS)], out_hbm.at[cid, sid])


@functools.partial(
    pl.kernel,
    out_type=jax.ShapeDtypeStruct((B, D), jnp.float32),
    mesh=_mesh,
    compiler_params=_SC_PARAMS,
    scratch_types=[
        pltpu.VMEM((_BPW,), jnp.int32),
        pltpu.VMEM((_BPW, D), jnp.float32),
        pltpu.SemaphoreType.DMA,
    ],
)
def _bgather_kernel(x_hbm, idx_hbm, out_hbm, idxv, rowsv, sem):
    wid = lax.axis_index("s") * _SC_CORES + lax.axis_index("c")
    base = pl.multiple_of(wid * _BPW, 8)
    pltpu.sync_copy(idx_hbm.at[pl.ds(base, _BPW)], idxv)
    pltpu.async_copy(x_hbm.at[idxv], rowsv, sem).wait()
    pltpu.sync_copy(rowsv, out_hbm.at[pl.ds(base, _BPW)])


def _ln(x, g, b):
    mu = jnp.mean(x, axis=-1, keepdims=True)
    v = jnp.mean((x - mu) ** 2, axis=-1, keepdims=True)
    return (x - mu) / jnp.sqrt(v + 1e-5) * g + b


def _dinv_body(parts_ref, out_ref):
    deg = 1.0 + jnp.sum(parts_ref[...], axis=0, keepdims=True)
    out_ref[...] = lax.rsqrt(deg)


def _encoder_body(census_ref, pop_ref, w1a_ref, w1b_ref, b1_ref, g1_ref,
                  be1_ref, w2_ref, b2_ref, g2_ref, be2_ref, out_ref):
    other = jnp.sum(pop_ref[...], axis=1, keepdims=True)
    h = (jnp.dot(census_ref[...], w1a_ref[...], precision=_HIGH)
         + other * w1b_ref[...] + b1_ref[...])
    h = jax.nn.relu(_ln(h, g1_ref[...], be1_ref[...]))
    h = jnp.dot(h, w2_ref[...], precision=_HIGH) + b2_ref[...]
    out_ref[...] = jax.nn.relu(_ln(h, g2_ref[...], be2_ref[...]))


def _pre_body(x_ref, w_ref, dinv_ref, xw_ref, xws_ref):
    xw = jnp.dot(x_ref[...], w_ref[...], precision=_HIGH)
    xw_ref[...] = xw
    xws_ref[...] = xw * dinv_ref[...]


def _post_body(parts_ref, xw_ref, dinv_ref, b_ref, g_ref, be_ref, xin_ref,
               out_ref, *, residual):
    dinv = dinv_ref[...]
    agg = ((parts_ref[0] + parts_ref[1]) * dinv
           + xw_ref[...] * (dinv * dinv) + b_ref[...])
    y = jax.nn.relu(_ln(agg, g_ref[...], be_ref[...]))
    if residual:
        y = y + xin_ref[...]
    out_ref[...] = y


def _pred_body(bs_ref, eth_ref, table_ref, w1a_ref, w1b_ref, b1_ref, g1_ref,
               be1_ref, w2_ref, b2_ref, g2_ref, be2_ref, w3_ref, b3_ref,
               out_ref):
    onehot = (lax.broadcasted_iota(jnp.int32, (B, NE), 1)
              == eth_ref[...]).astype(jnp.float32)
    ee = jnp.dot(onehot, table_ref[...], precision=_HIGH)
    c1 = (jnp.dot(bs_ref[...], w1a_ref[...], precision=_HIGH)
          + jnp.dot(ee, w1b_ref[...], precision=_HIGH) + b1_ref[...])
    p = jax.nn.relu(_ln(c1, g1_ref[...], be1_ref[...]))
    p = jnp.dot(p, w2_ref[...], precision=_HIGH) + b2_ref[...]
    p = jax.nn.relu(_ln(p, g2_ref[...], be2_ref[...]))
    p = jnp.dot(p, w3_ref[...], precision=_HIGH) + b3_ref[...]
    out_ref[...] = jax.nn.softplus(p)


def _f32(shape):
    return jax.ShapeDtypeStruct(shape, jnp.float32)


_dinv_call = pl.pallas_call(_dinv_body, out_shape=_f32((1, N)))
_encoder_call = pl.pallas_call(_encoder_body, out_shape=_f32((N, D)))
_pre_call = pl.pallas_call(_pre_body, out_shape=(_f32((N, D)), _f32((N, D))))
_post_res_call = pl.pallas_call(
    functools.partial(_post_body, residual=True), out_shape=_f32((N, D)))
_post_nores_call = pl.pallas_call(
    functools.partial(_post_body, residual=False), out_shape=_f32((N, D)))
_pred_call = pl.pallas_call(_pred_body, out_shape=_f32((B, 1)))


def kernel(full_graph_census, full_graph_population_t, ethnicity, node_idx,
           edge_index, edge_weight,
           enc_W1, enc_b1, enc_g1, enc_be1, enc_W2, enc_b2, enc_g2, enc_be2,
           gcn_W0, gcn_b0, gcn_g0, gcn_be0, gcn_W1, gcn_b1, gcn_g1, gcn_be1,
           gcn_W2, gcn_b2, gcn_g2, gcn_be2, eth_table,
           pred_W1, pred_b1, pred_g1, pred_be1, pred_W2, pred_b2, pred_g2,
           pred_be2, pred_W3, pred_b3):
    s = edge_index[0].astype(jnp.int32)
    d = edge_index[1].astype(jnp.int32)
    w = edge_weight
    npad = _EPAD - E
    pad_idx = (jnp.arange(npad, dtype=jnp.int32) * 97) % N
    sp = jnp.concatenate([s, pad_idx])
    dp = jnp.concatenate([d, pad_idx])
    wp = jnp.concatenate([w, jnp.zeros((npad,), jnp.float32)])

    row = lambda a: a.reshape(1, -1)

    deg_parts = _deg_kernel(d, w).reshape(_NW, N)
    dinv = _dinv_call(deg_parts).reshape(N, 1)

    h = _encoder_call(full_graph_census, full_graph_population_t,
                      enc_W1[:NC], enc_W1[NC:], row(enc_b1), row(enc_g1),
                      row(enc_be1), enc_W2, row(enc_b2), row(enc_g2),
                      row(enc_be2))

    layers = [
        (gcn_W0, gcn_b0, gcn_g0, gcn_be0),
        (gcn_W1, gcn_b1, gcn_g1, gcn_be1),
        (gcn_W2, gcn_b2, gcn_g2, gcn_be2),
    ]
    x = h
    for i, (Wt, bb, g, be) in enumerate(layers):
        xw, xws = _pre_call(x, Wt, dinv)
        parts = _agg_kernel(xws, sp, dp, wp).reshape(_SC_CORES, N, D)
        post = _post_res_call if i > 0 else _post_nores_call
        x = post(parts, xw, dinv, row(bb), row(g), row(be), x)

    bs = _bgather_kernel(x, node_idx.astype(jnp.int32))
    out = _pred_call(bs, ethnicity.astype(jnp.int32).reshape(B, 1), eth_table,
                     pred_W1[:D], pred_W1[D:], row(pred_b1), row(pred_g1),
                     row(pred_be1), pred_W2, row(pred_b2), row(pred_g2),
                     row(pred_be2), pred_W3, row(pred_b3))
    return out.reshape(B)


# speedup vs baseline: 18.3529x; 1.0598x over previous
import dataclasses
import functools

import jax
import jax.numpy as jnp
from jax import lax
from jax.experimental import pallas as pl
from jax.experimental.pallas import tpu as pltpu
from jax.experimental.pallas import tpu_sc as plsc

N = 10000
E = 320000
NC = 64
NE = 16
B = 4096
D = 128
ED = 32

_SC_CORES = 2
_SC_SUBCORES = 16
_NW = _SC_CORES * _SC_SUBCORES
_EPT = E // _NW
_W = 72
_NGW = 140
_EPTA = _NGW * _W
_EPAD = _NW * _EPTA
_RPS = N // _SC_SUBCORES
_ZR = 125
_BPW = B // _NW

_mesh = plsc.VectorSubcoreMesh(core_axis_name="c", subcore_axis_name="s")

_SC_PARAMS = pltpu.CompilerParams()
if "needs_layout_passes" in pltpu.CompilerParams.__dataclass_fields__:
    _SC_PARAMS = dataclasses.replace(_SC_PARAMS, needs_layout_passes=False)

_HIGH = jax.lax.Precision.HIGHEST


@functools.partial(
    pl.kernel,
    out_type=jax.ShapeDtypeStruct((_NW * N,), jnp.float32),
    mesh=_mesh,
    compiler_params=_SC_PARAMS,
    scratch_types=[
        pltpu.VMEM((_EPT,), jnp.int32),
        pltpu.VMEM((_EPT,), jnp.float32),
        pltpu.VMEM((N,), jnp.float32),
    ],
)
def _deg_kernel(d_hbm, w_hbm, out_hbm, didx, wvals, deg):
    cid = lax.axis_index("c")
    sid = lax.axis_index("s")
    wid = cid * _SC_SUBCORES + sid
    base = pl.multiple_of(wid * _EPT, 8)
    pltpu.sync_copy(d_hbm.at[pl.ds(base, _EPT)], didx)
    pltpu.sync_copy(w_hbm.at[pl.ds(base, _EPT)], wvals)

    @pl.loop(0, N, step=16)
    def _(i):
        deg[pl.ds(i, 16)] = jnp.zeros((16,), jnp.float32)

    @pl.loop(0, _EPT, step=16)
    def _(i):
        idx = didx[pl.ds(i, 16)]
        vals = wvals[pl.ds(i, 16)]
        plsc.addupdate_scatter(deg, [idx], vals)

    pltpu.sync_copy(deg, out_hbm.at[pl.ds(pl.multiple_of(wid * N, 8), N)])


_NBUF = 3


@functools.partial(
    pl.kernel,
    out_type=jax.ShapeDtypeStruct((_SC_CORES, _SC_SUBCORES, _RPS, D),
                                  jnp.float32),
    mesh=_mesh,
    compiler_params=_SC_PARAMS,
    scratch_types=[
        pltpu.VMEM((_EPTA,), jnp.int32),
        pltpu.VMEM((_EPTA,), jnp.float32),
        [pltpu.VMEM((_W,), jnp.int32) for _ in range(_NBUF)],
        [pltpu.VMEM((_W, D), jnp.float32) for _ in range(_NBUF)],
        pltpu.VMEM_SHARED((N, D), jnp.float32),
        [pltpu.SemaphoreType.DMA for _ in range(3 * _NBUF)],
    ],
)
def _agg_kernel(xws_hbm, s_hbm, d_hbm, w_hbm, out_hbm,
                sidx, wvals, dbufs, bufs, acc, sems):
    cid = lax.axis_index("c")
    sid = lax.axis_index("s")
    wid = cid * _SC_SUBCORES + sid
    base = pl.multiple_of(wid * _EPTA, 8)
    gsems = sems[:_NBUF]
    ssems = sems[_NBUF:2 * _NBUF]
    dsems = sems[2 * _NBUF:]

    @pl.loop(0, _W)
    def _(r):
        for c in range(D // 16):
            bufs[0][r, pl.ds(c * 16, 16)] = jnp.zeros((16,), jnp.float32)

    nz = _RPS // _W
    for k in range(nz):
        pltpu.sync_copy(bufs[0], acc.at[pl.ds(sid * _RPS + k * _W, _W)])
    pltpu.sync_copy(bufs[0].at[pl.ds(0, _RPS - nz * _W)],
                    acc.at[pl.ds(sid * _RPS + nz * _W, _RPS - nz * _W)])
    plsc.subcore_barrier()

    pltpu.sync_copy(s_hbm.at[pl.ds(base, _EPTA)], sidx)
    pltpu.sync_copy(w_hbm.at[pl.ds(base, _EPTA)], wvals)

    def start_gather(w, b):
        off = pl.multiple_of(w * _W, 8)
        pltpu.async_copy(xws_hbm.at[sidx.at[pl.ds(off, _W)]], bufs[b],
                         gsems[b])
        pltpu.async_copy(d_hbm.at[pl.ds(base + off, _W)], dbufs[b], dsems[b])

    def wait_gather(b):
        pltpu.make_async_copy(xws_hbm.at[pl.ds(0, _W)], bufs[b],
                              gsems[b]).wait()

    def scale(w, b):
        off = w * _W

        @pl.loop(0, _W)
        def _(e):
            wv = plsc.load_gather(wvals, [jnp.full((16,), off + e, jnp.int32)])
            for c in range(D // 16):
                sl = pl.ds(c * 16, 16)
                bufs[b][e, sl] = bufs[b][e, sl] * wv

    def start_scatter(w, b):
        pltpu.make_async_copy(d_hbm.at[pl.ds(0, _W)], dbufs[b],
                              dsems[b]).wait()
        pltpu.async_copy(bufs[b], acc.at[dbufs[b]], ssems[b], add=True)

    def wait_scatter(b):
        pltpu.make_async_copy(bufs[b], acc.at[pl.ds(0, _W)], ssems[b]).wait()

    def window(w, b, wait_prev, prefetch):
        wait_gather(b)
        scale(w, b)
        start_scatter(w, b)
        pb = (b - 1) % _NBUF
        if wait_prev:
            wait_scatter(pb)
        if prefetch:
            start_gather(w + _NBUF - 1, pb)

    _SE = _NGW - 4

    for j in range(_NBUF - 1):
        start_gather(j, j)
    window(0, 0, wait_prev=False, prefetch=True)

    @pl.loop(1, _SE, step=_NBUF)
    def _(w0):
        for j in range(_NBUF):
            window(w0 + j, (1 + j) % _NBUF, wait_prev=True, prefetch=True)

    for w in range(_SE, _NGW):
        window(w, w % _NBUF, wait_prev=True, prefetch=(w + _NBUF - 1 < _NGW))
    wait_scatter((_NGW - 1) % _NBUF)

    plsc.subcore_barrier()
    pltpu.sync_copy(acc.at[pl.ds(sid * _RPS, _RPS)], out_hbm.at[cid, sid])


@functools.partial(
    pl.kernel,
    out_type=jax.ShapeDtypeStruct((B, D), jnp.float32),
    mesh=_mesh,
    compiler_params=_SC_PARAMS,
    scratch_types=[
        pltpu.VMEM((_BPW,), jnp.int32),
        pltpu.VMEM((_BPW, D), jnp.float32),
        pltpu.SemaphoreType.DMA,
    ],
)
def _bgather_kernel(x_hbm, idx_hbm, out_hbm, idxv, rowsv, sem):
    wid = lax.axis_index("s") * _SC_CORES + lax.axis_index("c")
    base = pl.multiple_of(wid * _BPW, 8)
    pltpu.sync_copy(idx_hbm.at[pl.ds(base, _BPW)], idxv)
    pltpu.async_copy(x_hbm.at[idxv], rowsv, sem).wait()
    pltpu.sync_copy(rowsv, out_hbm.at[pl.ds(base, _BPW)])


def _ln(x, g, b):
    mu = jnp.mean(x, axis=-1, keepdims=True)
    v = jnp.mean((x - mu) ** 2, axis=-1, keepdims=True)
    return (x - mu) / jnp.sqrt(v + 1e-5) * g + b


def _dinv_body(parts_ref, out_ref):
    deg = 1.0 + jnp.sum(parts_ref[...], axis=0, keepdims=True)
    out_ref[...] = lax.rsqrt(deg)


def _encpre0_body(census_ref, pop_ref, w1a_ref, w1b_ref, b1_ref, g1_ref,
                  be1_ref, w2_ref, b2_ref, g2_ref, be2_ref, dinv_ref,
                  gw0_ref, xw_ref, xws_ref):
    other = jnp.sum(pop_ref[...], axis=1, keepdims=True)
    h = (jnp.dot(census_ref[...], w1a_ref[...], precision=_HIGH)
         + other * w1b_ref[...] + b1_ref[...])
    h = jax.nn.relu(_ln(h, g1_ref[...], be1_ref[...]))
    h = jnp.dot(h, w2_ref[...], precision=_HIGH) + b2_ref[...]
    h = jax.nn.relu(_ln(h, g2_ref[...], be2_ref[...]))
    xw = jnp.dot(h, gw0_ref[...], precision=_HIGH)
    xw_ref[...] = xw
    xws_ref[...] = xw * dinv_ref[...]


def _mid_body(parts_ref, xw_ref, dinv_ref, b_ref, g_ref, be_ref, xin_ref,
              wn_ref, x_ref, xwn_ref, xwsn_ref, *, residual):
    dinv = dinv_ref[...]
    agg = ((parts_ref[0] + parts_ref[1]) * dinv
           + xw_ref[...] * (dinv * dinv) + b_ref[...])
    y = jax.nn.relu(_ln(agg, g_ref[...], be_ref[...]))
    if residual:
        y = y + xin_ref[...]
    x_ref[...] = y
    xw = jnp.dot(y, wn_ref[...], precision=_HIGH)
    xwn_ref[...] = xw
    xwsn_ref[...] = xw * dinv


def _post_body(parts_ref, xw_ref, dinv_ref, b_ref, g_ref, be_ref, xin_ref,
               out_ref):
    dinv = dinv_ref[...]
    agg = ((parts_ref[0] + parts_ref[1]) * dinv
           + xw_ref[...] * (dinv * dinv) + b_ref[...])
    out_ref[...] = jax.nn.relu(_ln(agg, g_ref[...], be_ref[...])) + xin_ref[...]


def _pred_body(bs_ref, eth_ref, table_ref, w1a_ref, w1b_ref, b1_ref, g1_ref,
               be1_ref, w2_ref, b2_ref, g2_ref, be2_ref, w3_ref, b3_ref,
               out_ref):
    onehot = (lax.broadcasted_iota(jnp.int32, (B, NE), 1)
              == eth_ref[...]).astype(jnp.float32)
    ee = jnp.dot(onehot, table_ref[...], precision=_HIGH)
    c1 = (jnp.dot(bs_ref[...], w1a_ref[...], precision=_HIGH)
          + jnp.dot(ee, w1b_ref[...], precision=_HIGH) + b1_ref[...])
    p = jax.nn.relu(_ln(c1, g1_ref[...], be1_ref[...]))
    p = jnp.dot(p, w2_ref[...], precision=_HIGH) + b2_ref[...]
    p = jax.nn.relu(_ln(p, g2_ref[...], be2_ref[...]))
    p = jnp.dot(p, w3_ref[...], precision=_HIGH) + b3_ref[...]
    out_ref[...] = jax.nn.softplus(p)


def _f32(shape):
    return jax.ShapeDtypeStruct(shape, jnp.float32)


_BN = 2000
_NBK = N // _BN

def _rows(shape):
    return pl.BlockSpec((_BN,) + shape[1:], lambda i: (0,) * 0 + (i,) + (0,) * (len(shape) - 1))

_full = lambda shape: pl.BlockSpec(shape, lambda i: (0,) * len(shape))

_dinv_call = pl.pallas_call(_dinv_body, out_shape=_f32((1, N)))

_encpre0_call = pl.pallas_call(
    _encpre0_body,
    grid=(_NBK,),
    in_specs=[
        _rows((N, NC)), _rows((N, NE)),
        _full((NC, D)), _full((1, D)), _full((1, D)), _full((1, D)),
        _full((1, D)), _full((D, D)), _full((1, D)), _full((1, D)),
        _full((1, D)), _rows((N, 1)), _full((D, D)),
    ],
    out_specs=(_rows((N, D)), _rows((N, D))),
    out_shape=(_f32((N, D)), _f32((N, D))))

_parts_spec = pl.BlockSpec((_SC_CORES, _BN, D), lambda i: (0, i, 0))

def _mk_mid(residual):
    return pl.pallas_call(
        functools.partial(_mid_body, residual=residual),
        grid=(_NBK,),
        in_specs=[
            _parts_spec, _rows((N, D)), _rows((N, 1)),
            _full((1, D)), _full((1, D)), _full((1, D)),
            _rows((N, D)), _full((D, D)),
        ],
        out_specs=(_rows((N, D)), _rows((N, D)), _rows((N, D))),
        out_shape=(_f32((N, D)), _f32((N, D)), _f32((N, D))))

_mid_nores_call = _mk_mid(False)
_mid_res_call = _mk_mid(True)

_post_call = pl.pallas_call(
    _post_body,
    grid=(_NBK,),
    in_specs=[
        _parts_spec, _rows((N, D)), _rows((N, 1)),
        _full((1, D)), _full((1, D)), _full((1, D)), _rows((N, D)),
    ],
    out_specs=_rows((N, D)),
    out_shape=_f32((N, D)))
_pred_call = pl.pallas_call(_pred_body, out_shape=_f32((B, 1)))


def kernel(full_graph_census, full_graph_population_t, ethnicity, node_idx,
           edge_index, edge_weight,
           enc_W1, enc_b1, enc_g1, enc_be1, enc_W2, enc_b2, enc_g2, enc_be2,
           gcn_W0, gcn_b0, gcn_g0, gcn_be0, gcn_W1, gcn_b1, gcn_g1, gcn_be1,
           gcn_W2, gcn_b2, gcn_g2, gcn_be2, eth_table,
           pred_W1, pred_b1, pred_g1, pred_be1, pred_W2, pred_b2, pred_g2,
           pred_be2, pred_W3, pred_b3):
    s = edge_index[0].astype(jnp.int32)
    d = edge_index[1].astype(jnp.int32)
    w = edge_weight
    npad = _EPAD - E
    pad_idx = (jnp.arange(npad, dtype=jnp.int32) * 97) % N
    sp = jnp.concatenate([s, pad_idx])
    dp = jnp.concatenate([d, pad_idx])
    wp = jnp.concatenate([w, jnp.zeros((npad,), jnp.float32)])

    row = lambda a: a.reshape(1, -1)

    deg_parts = _deg_kernel(d, w).reshape(_NW, N)
    dinv = _dinv_call(deg_parts).reshape(N, 1)

    xw0, xws0 = _encpre0_call(
        full_graph_census, full_graph_population_t,
        enc_W1[:NC], enc_W1[NC:], row(enc_b1), row(enc_g1), row(enc_be1),
        enc_W2, row(enc_b2), row(enc_g2), row(enc_be2), dinv, gcn_W0)

    parts0 = _agg_kernel(xws0, sp, dp, wp).reshape(_SC_CORES, N, D)
    x1, xw1, xws1 = _mid_nores_call(parts0, xw0, dinv, row(gcn_b0),
                                    row(gcn_g0), row(gcn_be0), xw0, gcn_W1)
    parts1 = _agg_kernel(xws1, sp, dp, wp).reshape(_SC_CORES, N, D)
    x2, xw2, xws2 = _mid_res_call(parts1, xw1, dinv, row(gcn_b1),
                                  row(gcn_g1), row(gcn_be1), x1, gcn_W2)
    parts2 = _agg_kernel(xws2, sp, dp, wp).reshape(_SC_CORES, N, D)
    x3 = _post_call(parts2, xw2, dinv, row(gcn_b2), row(gcn_g2),
                    row(gcn_be2), x2)

    bs = _bgather_kernel(x3, node_idx.astype(jnp.int32))
    out = _pred_call(bs, ethnicity.astype(jnp.int32).reshape(B, 1), eth_table,
                     pred_W1[:D], pred_W1[D:], row(pred_b1), row(pred_g1),
                     row(pred_be1), pred_W2, row(pred_b2), row(pred_g2),
                     row(pred_be2), pred_W3, row(pred_b3))
    return out.reshape(B)
